# baseline (speedup 1.0000x reference)
import jax
import jax.numpy as jnp
from jax import lax
from jax.experimental import pallas as pl
from jax.experimental.pallas import tpu as pltpu
from jax.experimental.pallas import tpu_sc as plsc

_B = 16384
_V = 1000
_NC = 2
_NS = 16
_L = 16
_NG = 16
_CB = _B // _NG
_RC = 56
_NBUF = 2


def _onehot_body(x_hbm, out_hbm, x_v, *bufs_sems):
    bufs = bufs_sems[:_NBUF]
    sems = bufs_sems[_NBUF:]

    wid = lax.axis_index("s") * _NC + lax.axis_index("c")
    grp = wid % _NG
    half = wid // _NG
    b0 = grp * _CB

    pltpu.sync_copy(x_hbm.at[pl.ds(b0, _CB)], x_v)

    zrow = jnp.zeros((_L,), jnp.float32)

    def _zf(buf):
        @pl.loop(0, _RC)
        def _zf_r(r):
            for c in range(0, _CB, _L):
                buf[r, pl.ds(c, _L)] = zrow

    ones = jnp.ones((_L,), jnp.float32)
    zeros = jnp.zeros((_L,), jnp.float32)
    lane = lax.iota(jnp.int32, _L)

    def _scatter(buf, c0, nc, val, c0_prev=None, nc_prev=None):
        @pl.loop(0, _CB // _L, unroll=4)
        def _sc_j(j):
            xv = x_v[pl.ds(j * _L, _L)]
            colv = lane + j * _L
            if c0_prev is not None:
                maskp = (xv >= c0_prev) & (xv < c0_prev + nc_prev)
                plsc.store_scatter(
                    buf, [xv - c0_prev, colv], zeros, mask=maskp
                )
            mask = (xv >= c0) & (xv < c0 + nc)
            plsc.store_scatter(buf, [xv - c0, colv], val, mask=mask)

    cbase = half * 504
    handles = [None] * _NBUF
    prev = [None] * _NBUF
    for ch in range(8):
        b = ch % _NBUF
        c0 = cbase + ch * _RC
        if handles[b] is not None:
            handles[b].wait()
            _scatter(bufs[b], c0, _RC, ones, prev[b], _RC)
        else:
            _zf(bufs[b])
            _scatter(bufs[b], c0, _RC, ones)
        handles[b] = pltpu.async_copy(
            bufs[b], out_hbm.at[pl.ds(c0, _RC), pl.ds(b0, _CB)], sems[b]
        )
        prev[b] = c0
    handles[0].wait()
    c0 = cbase + 8 * _RC
    nc = jnp.where(half == 0, _RC, 48)
    _scatter(bufs[0], c0, nc, ones, prev[0], _RC)

    @pl.when(half == 0)
    def _last0():
        pltpu.sync_copy(bufs[0], out_hbm.at[pl.ds(c0, _RC), pl.ds(b0, _CB)])

    @pl.when(half == 1)
    def _last1():
        pltpu.sync_copy(
            bufs[0].at[pl.ds(0, 48)],
            out_hbm.at[pl.ds(c0, 48), pl.ds(b0, _CB)],
        )

    handles[1].wait()


def kernel(x):
    mesh = plsc.VectorSubcoreMesh(core_axis_name="c", subcore_axis_name="s")
    run = pl.kernel(
        _onehot_body,
        out_type=jax.ShapeDtypeStruct((_V, _B), jnp.float32),
        mesh=mesh,
        compiler_params=pltpu.CompilerParams(
            needs_layout_passes=False, use_tc_tiling_on_sc=True
        ),
        scratch_types=(
            [pltpu.VMEM((_CB,), jnp.int32)]
            + [pltpu.VMEM((_RC, _CB), jnp.float32)] * _NBUF
            + [pltpu.SemaphoreType.DMA] * _NBUF
        ),
    )
    return run(x.astype(jnp.int32)).T
